# Initial kernel scaffold; baseline (speedup 1.0000x reference)
import jax
import jax.numpy as jnp
from jax.experimental import pallas as pl


def kernel(feas, edge_index, W_self0, W_neigh0, b0, W_self1, W_neigh1, b1, W_lin, b_lin):
    raise NotImplementedError("write your pallas kernel here")


# speedup vs baseline: 13.8745x; 13.8745x over previous
import functools

import jax
import jax.numpy as jnp
from jax import lax
from jax.experimental import pallas as pl
from jax.experimental.pallas import tpu as pltpu
from jax.experimental.pallas import tpu_sc as plsc

N = 10000
E = 320000
D = 128
H0 = 64
H1 = 32
C = 47

NC = 2
NS = 16
NW = NC * NS
EPT = E // NW
CH = 80
NCHUNK = EPT // CH
K = 5
NGROUP = NCHUNK // K
NPAD = 10240
RPT = NPAD // NS
ZR = 80


def _scatter_body(with_deg, h, p_hbm, src_hbm, dst_hbm, *refs):
    if with_deg:
        out_hbm, deg_hbm, src_v, dst_v, rows, zbuf, acc, sem, ones_v, zbufd, dacc = refs
    else:
        out_hbm, src_v, dst_v, rows, zbuf, acc, sem = refs
    cid = lax.axis_index("c")
    sid = lax.axis_index("s")
    wid = sid * NC + cid

    def _zero_row(r, _):
        for cb in range(h // 16):
            zbuf[r, pl.ds(cb * 16, 16)] = jnp.zeros((16,), jnp.float32)
        if with_deg:
            zbufd[r, pl.ds(0, 16)] = jnp.zeros((16,), jnp.float32)
            ones_v[r, pl.ds(0, 16)] = jnp.ones((16,), jnp.float32)
        return 0

    lax.fori_loop(0, ZR, _zero_row, 0)
    for i in range(RPT // ZR):
        pltpu.sync_copy(zbuf, acc.at[pl.ds(sid * RPT + i * ZR, ZR)])
        if with_deg:
            pltpu.sync_copy(zbufd, dacc.at[pl.ds(sid * RPT + i * ZR, ZR)])
    plsc.subcore_barrier()

    pltpu.sync_copy(src_hbm.at[wid], src_v)
    pltpu.sync_copy(dst_hbm.at[wid], dst_v)

    def _group(g, _):
        descs = []
        for b in range(K):
            j = g * K + b
            descs.append(pltpu.async_copy(p_hbm.at[src_v.at[j]], rows.at[b], sem))
        for d in descs:
            d.wait()
        for b in range(K):
            j = g * K + b
            pltpu.sync_copy(rows.at[b], acc.at[dst_v.at[j]], add=True)
            if with_deg:
                pltpu.sync_copy(ones_v, dacc.at[dst_v.at[j]], add=True)
        return 0

    lax.fori_loop(0, NGROUP, _group, 0)
    plsc.subcore_barrier()

    pltpu.sync_copy(acc.at[pl.ds(sid * RPT, RPT)],
                    out_hbm.at[cid, pl.ds(sid * RPT, RPT)])
    if with_deg:
        pltpu.sync_copy(dacc.at[pl.ds(sid * RPT, RPT)],
                        deg_hbm.at[cid, pl.ds(sid * RPT, RPT)])


def _make_scatter(h, with_deg):
    mesh = plsc.VectorSubcoreMesh(
        core_axis_name="c", subcore_axis_name="s", num_cores=NC, num_subcores=NS)
    out_type = [jax.ShapeDtypeStruct((NC, NPAD, h), jnp.float32)]
    scratch = [
        pltpu.VMEM((NCHUNK, CH), jnp.int32),
        pltpu.VMEM((NCHUNK, CH), jnp.int32),
        pltpu.VMEM((K, CH, h), jnp.float32),
        pltpu.VMEM((ZR, h), jnp.float32),
        pltpu.VMEM_SHARED((NPAD, h), jnp.float32),
        pltpu.SemaphoreType.DMA,
    ]
    if with_deg:
        out_type.append(jax.ShapeDtypeStruct((NC, NPAD, 16), jnp.float32))
        scratch += [
            pltpu.VMEM((CH, 16), jnp.float32),
            pltpu.VMEM((ZR, 16), jnp.float32),
            pltpu.VMEM_SHARED((NPAD, 16), jnp.float32),
        ]
    return pl.kernel(
        functools.partial(_scatter_body, with_deg, h),
        out_type=out_type, mesh=mesh, scratch_types=scratch,
        compiler_params=pltpu.CompilerParams(use_tc_tiling_on_sc=False))


def _proj_body(x_ref, w_ref, o_ref):
    o_ref[...] = jnp.dot(x_ref[...], w_ref[...], preferred_element_type=jnp.float32)


def _layer0_body(feas_ref, part_ref, degp_ref, ws0_ref, wn1_ref, b0_ref,
                 h0_ref, p1_ref):
    deg = degp_ref[0] + degp_ref[1]
    inv = 1.0 / jnp.maximum(deg[:N, 0:1], 1.0)
    agg = (part_ref[0] + part_ref[1])[:N] * inv
    h0 = jnp.dot(feas_ref[...], ws0_ref[...], preferred_element_type=jnp.float32)
    h0 = jnp.maximum(h0 + agg + b0_ref[...], 0.0)
    h0_ref[...] = h0
    p1_ref[...] = jnp.dot(h0, wn1_ref[...], preferred_element_type=jnp.float32)


def _layer1_body(h0_ref, part_ref, degp_ref, ws1_ref, wlin_ref, b1_ref,
                 blin_ref, out_ref, h1_ref):
    deg = degp_ref[0] + degp_ref[1]
    inv = 1.0 / jnp.maximum(deg[:N, 0:1], 1.0)
    agg = (part_ref[0] + part_ref[1])[:N] * inv
    h1 = jnp.dot(h0_ref[...], ws1_ref[...], preferred_element_type=jnp.float32)
    h1 = h1 + agg + b1_ref[...]
    h1_ref[...] = h1
    out_ref[...] = jnp.dot(h1, wlin_ref[...], preferred_element_type=jnp.float32) + blin_ref[...]


def kernel(feas, edge_index, W_self0, W_neigh0, b0, W_self1, W_neigh1, b1,
           W_lin, b_lin):
    src3 = edge_index[0].astype(jnp.int32).reshape(NW, NCHUNK, CH)
    dst3 = edge_index[1].astype(jnp.int32).reshape(NW, NCHUNK, CH)

    p0 = pl.pallas_call(
        _proj_body,
        out_shape=jax.ShapeDtypeStruct((N, H0), jnp.float32),
    )(feas, W_neigh0)

    part0, degp = _make_scatter(H0, True)(p0, src3, dst3)

    h0, p1 = pl.pallas_call(
        _layer0_body,
        out_shape=[jax.ShapeDtypeStruct((N, H0), jnp.float32),
                   jax.ShapeDtypeStruct((N, H1), jnp.float32)],
    )(feas, part0, degp, W_self0, W_neigh1, b0.reshape(1, H0))

    (part1,) = _make_scatter(H1, False)(p1, src3, dst3)

    out, h1 = pl.pallas_call(
        _layer1_body,
        out_shape=[jax.ShapeDtypeStruct((N, C), jnp.float32),
                   jax.ShapeDtypeStruct((N, H1), jnp.float32)],
    )(h0, part1, degp, W_self1, W_lin, b1.reshape(1, H1), b_lin.reshape(1, C))

    return (out, h1)


# speedup vs baseline: 17.5452x; 1.2646x over previous
import functools

import jax
import jax.numpy as jnp
from jax import lax
from jax.experimental import pallas as pl
from jax.experimental.pallas import tpu as pltpu
from jax.experimental.pallas import tpu_sc as plsc

N = 10000
E = 320000
D = 128
H0 = 64
H1 = 32
C = 47

NC = 2
NS = 16
NW = NC * NS
EPT = E // NW
CH = 80
NCHUNK = EPT // CH
K = 5
NGROUP = NCHUNK // K
NPAD = 10240
RPT = NPAD // NS
ZR = 80


def _scatter_body(with_deg, h, p_hbm, src_hbm, dst_hbm, *refs):
    if with_deg:
        (out_hbm, deg_hbm, src_v, dst_v, rows, zbuf, acc, gsem, ssem,
         ones_v, zbufd, dacc) = refs
    else:
        out_hbm, src_v, dst_v, rows, zbuf, acc, gsem, ssem = refs
    cid = lax.axis_index("c")
    sid = lax.axis_index("s")
    wid = sid * NC + cid

    def _zero_row(r, _):
        for cb in range(h // 16):
            zbuf[r, pl.ds(cb * 16, 16)] = jnp.zeros((16,), jnp.float32)
        if with_deg:
            zbufd[r, pl.ds(0, 16)] = jnp.zeros((16,), jnp.float32)
            ones_v[r, pl.ds(0, 16)] = jnp.ones((16,), jnp.float32)
        return 0

    lax.fori_loop(0, ZR, _zero_row, 0)
    for i in range(RPT // ZR):
        pltpu.sync_copy(zbuf, acc.at[pl.ds(sid * RPT + i * ZR, ZR)])
        if with_deg:
            pltpu.sync_copy(zbufd, dacc.at[pl.ds(sid * RPT + i * ZR, ZR)])
    plsc.subcore_barrier()

    pltpu.sync_copy(src_hbm.at[wid], src_v)
    pltpu.sync_copy(dst_hbm.at[wid], dst_v)

    def _fire_gathers(g, bank):
        for b in range(K):
            pltpu.async_copy(p_hbm.at[src_v.at[g * K + b]],
                             rows.at[bank * K + b], gsem)

    def _drain_gathers(g, bank):
        for b in range(K):
            pltpu.make_async_copy(p_hbm.at[src_v.at[g * K + b]],
                                  rows.at[bank * K + b], gsem).wait()

    _fire_gathers(0, 0)

    def _group(g, _):
        bank = lax.rem(g, 2)
        _drain_gathers(g, bank)

        @pl.when(g < NGROUP - 1)
        def _():
            _fire_gathers(g + 1, 1 - bank)

        descs = []
        for b in range(K):
            j = g * K + b
            descs.append(pltpu.async_copy(rows.at[bank * K + b],
                                          acc.at[dst_v.at[j]], ssem, add=True))
            if with_deg:
                descs.append(pltpu.async_copy(ones_v, dacc.at[dst_v.at[j]],
                                              ssem, add=True))
        for d in descs:
            d.wait()
        return 0

    lax.fori_loop(0, NGROUP, _group, 0)
    plsc.subcore_barrier()

    pltpu.sync_copy(acc.at[pl.ds(sid * RPT, RPT)],
                    out_hbm.at[cid, pl.ds(sid * RPT, RPT)])
    if with_deg:
        pltpu.sync_copy(dacc.at[pl.ds(sid * RPT, RPT)],
                        deg_hbm.at[cid, pl.ds(sid * RPT, RPT)])


def _make_scatter(h, with_deg):
    mesh = plsc.VectorSubcoreMesh(
        core_axis_name="c", subcore_axis_name="s", num_cores=NC, num_subcores=NS)
    out_type = [jax.ShapeDtypeStruct((NC, NPAD, h), jnp.float32)]
    scratch = [
        pltpu.VMEM((NCHUNK, CH), jnp.int32),
        pltpu.VMEM((NCHUNK, CH), jnp.int32),
        pltpu.VMEM((2 * K, CH, h), jnp.float32),
        pltpu.VMEM((ZR, h), jnp.float32),
        pltpu.VMEM_SHARED((NPAD, h), jnp.float32),
        pltpu.SemaphoreType.DMA,
        pltpu.SemaphoreType.DMA,
    ]
    if with_deg:
        out_type.append(jax.ShapeDtypeStruct((NC, NPAD, 16), jnp.float32))
        scratch += [
            pltpu.VMEM((CH, 16), jnp.float32),
            pltpu.VMEM((ZR, 16), jnp.float32),
            pltpu.VMEM_SHARED((NPAD, 16), jnp.float32),
        ]
    return pl.kernel(
        functools.partial(_scatter_body, with_deg, h),
        out_type=out_type, mesh=mesh, scratch_types=scratch,
        compiler_params=pltpu.CompilerParams(use_tc_tiling_on_sc=False))


def _proj_body(x_ref, w_ref, o_ref):
    o_ref[...] = jnp.dot(x_ref[...], w_ref[...], preferred_element_type=jnp.float32)


def _layer0_body(feas_ref, part_ref, degp_ref, ws0_ref, wn1_ref, b0_ref,
                 h0_ref, p1_ref):
    deg = degp_ref[0] + degp_ref[1]
    inv = 1.0 / jnp.maximum(deg[:N, 0:1], 1.0)
    agg = (part_ref[0] + part_ref[1])[:N] * inv
    h0 = jnp.dot(feas_ref[...], ws0_ref[...], preferred_element_type=jnp.float32)
    h0 = jnp.maximum(h0 + agg + b0_ref[...], 0.0)
    h0_ref[...] = h0
    p1_ref[...] = jnp.dot(h0, wn1_ref[...], preferred_element_type=jnp.float32)


def _layer1_body(h0_ref, part_ref, degp_ref, ws1_ref, wlin_ref, b1_ref,
                 blin_ref, out_ref, h1_ref):
    deg = degp_ref[0] + degp_ref[1]
    inv = 1.0 / jnp.maximum(deg[:N, 0:1], 1.0)
    agg = (part_ref[0] + part_ref[1])[:N] * inv
    h1 = jnp.dot(h0_ref[...], ws1_ref[...], preferred_element_type=jnp.float32)
    h1 = h1 + agg + b1_ref[...]
    h1_ref[...] = h1
    out_ref[...] = jnp.dot(h1, wlin_ref[...], preferred_element_type=jnp.float32) + blin_ref[...]


def kernel(feas, edge_index, W_self0, W_neigh0, b0, W_self1, W_neigh1, b1,
           W_lin, b_lin):
    src3 = edge_index[0].astype(jnp.int32).reshape(NW, NCHUNK, CH)
    dst3 = edge_index[1].astype(jnp.int32).reshape(NW, NCHUNK, CH)

    p0 = pl.pallas_call(
        _proj_body,
        out_shape=jax.ShapeDtypeStruct((N, H0), jnp.float32),
    )(feas, W_neigh0)

    part0, degp = _make_scatter(H0, True)(p0, src3, dst3)

    h0, p1 = pl.pallas_call(
        _layer0_body,
        out_shape=[jax.ShapeDtypeStruct((N, H0), jnp.float32),
                   jax.ShapeDtypeStruct((N, H1), jnp.float32)],
    )(feas, part0, degp, W_self0, W_neigh1, b0.reshape(1, H0))

    (part1,) = _make_scatter(H1, False)(p1, src3, dst3)

    out, h1 = pl.pallas_call(
        _layer1_body,
        out_shape=[jax.ShapeDtypeStruct((N, C), jnp.float32),
                   jax.ShapeDtypeStruct((N, H1), jnp.float32)],
    )(h0, part1, degp, W_self1, W_lin, b1.reshape(1, H1), b_lin.reshape(1, C))

    return (out, h1)


# speedup vs baseline: 19.8183x; 1.1296x over previous
import functools

import jax
import jax.numpy as jnp
from jax import lax
from jax.experimental import pallas as pl
from jax.experimental.pallas import tpu as pltpu
from jax.experimental.pallas import tpu_sc as plsc

N = 10000
E = 320000
D = 128
H0 = 64
H1 = 32
C = 47

NC = 2
NS = 16
NW = NC * NS
EPT = E // NW
CH = 80
NCHUNK = EPT // CH
K = 5
NGROUP = NCHUNK // K
NPAD = 10240
RPT = NPAD // NS
ZR = 80


def _scatter_body(with_deg, h, p_hbm, edge_hbm, *refs):
    if with_deg:
        (out_hbm, deg_hbm, src_v, dst_v, rows, zbuf, acc, gsem, ssem,
         ones_v, zbufd, dacc) = refs
    else:
        out_hbm, src_v, dst_v, rows, zbuf, acc, gsem, ssem = refs
    cid = lax.axis_index("c")
    sid = lax.axis_index("s")
    wid = sid * NC + cid

    def _zero_row(r, _):
        for cb in range(h // 16):
            zbuf[r, pl.ds(cb * 16, 16)] = jnp.zeros((16,), jnp.float32)
        if with_deg:
            zbufd[r, pl.ds(0, 16)] = jnp.zeros((16,), jnp.float32)
            ones_v[r, pl.ds(0, 16)] = jnp.ones((16,), jnp.float32)
        return 0

    lax.fori_loop(0, ZR, _zero_row, 0)
    for i in range(RPT // ZR):
        pltpu.sync_copy(zbuf, acc.at[pl.ds(sid * RPT + i * ZR, ZR)])
        if with_deg:
            pltpu.sync_copy(zbufd, dacc.at[pl.ds(sid * RPT + i * ZR, ZR)])
    plsc.subcore_barrier()

    pltpu.sync_copy(edge_hbm.at[0, wid], src_v)
    pltpu.sync_copy(edge_hbm.at[1, wid], dst_v)

    def _fire_gathers(g, bank):
        for b in range(K):
            pltpu.async_copy(p_hbm.at[src_v.at[g * K + b]],
                             rows.at[bank * K + b], gsem)

    def _drain_gathers(g, bank):
        for b in range(K):
            pltpu.make_async_copy(p_hbm.at[src_v.at[g * K + b]],
                                  rows.at[bank * K + b], gsem).wait()

    _fire_gathers(0, 0)

    def _group(g, _):
        bank = lax.rem(g, 2)
        _drain_gathers(g, bank)

        @pl.when(g < NGROUP - 1)
        def _():
            _fire_gathers(g + 1, 1 - bank)

        descs = []
        for b in range(K):
            j = g * K + b
            descs.append(pltpu.async_copy(rows.at[bank * K + b],
                                          acc.at[dst_v.at[j]], ssem, add=True))
            if with_deg:
                descs.append(pltpu.async_copy(ones_v, dacc.at[dst_v.at[j]],
                                              ssem, add=True))
        for d in descs:
            d.wait()
        return 0

    lax.fori_loop(0, NGROUP, _group, 0)
    plsc.subcore_barrier()

    pltpu.sync_copy(acc.at[pl.ds(sid * RPT, RPT)],
                    out_hbm.at[pl.ds(sid * RPT, RPT), pl.ds(cid * h, h)])
    if with_deg:
        pltpu.sync_copy(dacc.at[pl.ds(sid * RPT, RPT)],
                        deg_hbm.at[pl.ds(sid * RPT, RPT), pl.ds(cid * 64, 16)])


def _make_scatter(h, with_deg):
    mesh = plsc.VectorSubcoreMesh(
        core_axis_name="c", subcore_axis_name="s", num_cores=NC, num_subcores=NS)
    out_type = [jax.ShapeDtypeStruct((NPAD, NC * h), jnp.float32)]
    scratch = [
        pltpu.VMEM((NCHUNK, CH), jnp.int32),
        pltpu.VMEM((NCHUNK, CH), jnp.int32),
        pltpu.VMEM((2 * K, CH, h), jnp.float32),
        pltpu.VMEM((ZR, h), jnp.float32),
        pltpu.VMEM_SHARED((NPAD, h), jnp.float32),
        pltpu.SemaphoreType.DMA,
        pltpu.SemaphoreType.DMA,
    ]
    if with_deg:
        out_type.append(jax.ShapeDtypeStruct((NPAD, 128), jnp.float32))
        scratch += [
            pltpu.VMEM((CH, 16), jnp.float32),
            pltpu.VMEM((ZR, 16), jnp.float32),
            pltpu.VMEM_SHARED((NPAD, 16), jnp.float32),
        ]
    return pl.kernel(
        functools.partial(_scatter_body, with_deg, h),
        out_type=out_type, mesh=mesh, scratch_types=scratch,
        compiler_params=pltpu.CompilerParams(use_tc_tiling_on_sc=False))


def _proj_body(x_ref, w_ref, o_ref):
    o_ref[...] = jnp.dot(x_ref[...], w_ref[...], preferred_element_type=jnp.float32)


def _inv_deg(degp_ref):
    deg = degp_ref[:, 0:1] + degp_ref[:, 64:65]
    return 1.0 / jnp.maximum(deg[:N], 1.0)


def _layer0_body(feas_ref, part_ref, degp_ref, ws0_ref, wn1_ref,
                 b0_ref, h0_ref, p1_ref):
    inv = _inv_deg(degp_ref)
    agg = (part_ref[:, :H0] + part_ref[:, H0:])[:N] * inv
    h0 = jnp.dot(feas_ref[...], ws0_ref[...], preferred_element_type=jnp.float32)
    h0 = jnp.maximum(h0 + agg + b0_ref[...], 0.0)
    h0_ref[...] = h0
    p1_ref[...] = jnp.dot(h0, wn1_ref[...], preferred_element_type=jnp.float32)


def _layer1_body(h0_ref, part_ref, degp_ref, ws1_ref, wlin_ref,
                 b1_ref, blin_ref, out_ref, h1_ref):
    inv = _inv_deg(degp_ref)
    agg = (part_ref[:, :H1] + part_ref[:, H1:])[:N] * inv
    h1 = jnp.dot(h0_ref[...], ws1_ref[...], preferred_element_type=jnp.float32)
    h1 = h1 + agg + b1_ref[...]
    h1_ref[...] = h1
    out_ref[...] = jnp.dot(h1, wlin_ref[...], preferred_element_type=jnp.float32) + blin_ref[...]


def kernel(feas, edge_index, W_self0, W_neigh0, b0, W_self1, W_neigh1, b1,
           W_lin, b_lin):
    edge_r = edge_index.astype(jnp.int32).reshape(2, NW, NCHUNK, CH)

    p0 = pl.pallas_call(
        _proj_body,
        out_shape=jax.ShapeDtypeStruct((N, H0), jnp.float32),
    )(feas, W_neigh0)

    part0, degp = _make_scatter(H0, True)(p0, edge_r)

    h0, p1 = pl.pallas_call(
        _layer0_body,
        out_shape=[jax.ShapeDtypeStruct((N, H0), jnp.float32),
                   jax.ShapeDtypeStruct((N, H1), jnp.float32)],
    )(feas, part0, degp, W_self0, W_neigh1, b0.reshape(1, H0))

    (part1,) = _make_scatter(H1, False)(p1, edge_r)

    out, h1 = pl.pallas_call(
        _layer1_body,
        out_shape=[jax.ShapeDtypeStruct((N, C), jnp.float32),
                   jax.ShapeDtypeStruct((N, H1), jnp.float32)],
    )(h0, part1, degp, W_self1, W_lin, b1.reshape(1, H1),
      b_lin.reshape(1, C))

    return (out, h1)
